# baseline (speedup 1.0000x reference)
import functools

import jax
import jax.numpy as jnp
from jax import lax
from jax.experimental import pallas as pl
from jax.experimental.pallas import tpu as pltpu
from jax.experimental.pallas import tpu_sc as plsc

_NC, _NS, _L = 2, 16, 16
_NW = _NC * _NS
_CH = 8


def _sc_body(x_hbm, tab_hbm, o_hbm, xbuf, tbuf):
    B, S, D = 4, 2048, 4096
    rows_per_w = S // _NW
    chunk_elems = _CH * D
    wid = lax.axis_index("c") * _NS + lax.axis_index("s")
    s_base = wid * rows_per_w

    for c in range(rows_per_w // _CH):
        s0 = s_base + c * _CH
        pltpu.sync_copy(tab_hbm.at[pl.ds(s0 * D, chunk_elems)], tbuf)
        for b in range(B):
            off = (b * S + s0) * D
            pltpu.sync_copy(x_hbm.at[pl.ds(off, chunk_elems)], xbuf)

            def add_body(g, _):
                plsc.addupdate(xbuf.at[pl.ds(g * _L, _L)], tbuf[pl.ds(g * _L, _L)])
                return 0

            lax.fori_loop(0, chunk_elems // _L, add_body, 0, unroll=8)
            pltpu.sync_copy(xbuf, o_hbm.at[pl.ds(off, chunk_elems)])


def kernel(x, pos_table):
    B, S, D = x.shape
    out_flat = pl.kernel(
        _sc_body,
        out_type=jax.ShapeDtypeStruct((B * S * D,), x.dtype),
        mesh=plsc.VectorSubcoreMesh(core_axis_name="c", subcore_axis_name="s"),
        scratch_types=[
            pltpu.VMEM((_CH * D,), jnp.float32),
            pltpu.VMEM((_CH * D,), jnp.float32),
        ],
    )(x.reshape(-1), pos_table.reshape(-1))
    return out_flat.reshape(B, S, D)


# speedup vs baseline: 1.1322x; 1.1322x over previous
import jax
import jax.numpy as jnp
from jax import lax
from jax.experimental import pallas as pl
from jax.experimental.pallas import tpu as pltpu
from jax.experimental.pallas import tpu_sc as plsc

_NC, _NS, _L = 2, 16, 16
_NW = _NC * _NS
_CH = 8
_B, _S, _D = 4, 2048, 4096


def _sc_body(x_hbm, tab_hbm, o_hbm, xb0, xb1, tbuf, ls0, ls1, ss0, ss1):
    rows_per_w = _S // _NW
    chunk = _CH * _D
    n_steps = (rows_per_w // _CH) * _B
    wid = lax.axis_index("c") * _NS + lax.axis_index("s")
    s_base = wid * rows_per_w

    xbufs = (xb0, xb1)
    lsems = (ls0, ls1)
    ssems = (ss0, ss1)

    def x_off(t):
        c, b = t // _B, t % _B
        return (b * _S + s_base + c * _CH) * _D

    def load(t):
        return pltpu.make_async_copy(
            x_hbm.at[pl.ds(x_off(t), chunk)], xbufs[t % 2], lsems[t % 2])

    def store(t):
        return pltpu.make_async_copy(
            xbufs[t % 2], o_hbm.at[pl.ds(x_off(t), chunk)], ssems[t % 2])

    load(0).start()
    for t in range(n_steps):
        if t + 1 < n_steps:
            if t >= 1:
                store(t - 1).wait()
            load(t + 1).start()
        if t % _B == 0:
            pltpu.sync_copy(
                tab_hbm.at[pl.ds((s_base + (t // _B) * _CH) * _D, chunk)], tbuf)
        load(t).wait()

        def add_body(g, _):
            plsc.addupdate(xbufs[t % 2].at[pl.ds(g * _L, _L)],
                           tbuf[pl.ds(g * _L, _L)])
            return 0

        lax.fori_loop(0, chunk // _L, add_body, 0, unroll=8)
        store(t).start()
    store(n_steps - 2).wait()
    store(n_steps - 1).wait()


def kernel(x, pos_table):
    B, S, D = x.shape
    out_flat = pl.kernel(
        _sc_body,
        out_type=jax.ShapeDtypeStruct((B * S * D,), x.dtype),
        mesh=plsc.VectorSubcoreMesh(core_axis_name="c", subcore_axis_name="s"),
        scratch_types=[
            pltpu.VMEM((_CH * _D,), jnp.float32),
            pltpu.VMEM((_CH * _D,), jnp.float32),
            pltpu.VMEM((_CH * _D,), jnp.float32),
            pltpu.SemaphoreType.DMA,
            pltpu.SemaphoreType.DMA,
            pltpu.SemaphoreType.DMA,
            pltpu.SemaphoreType.DMA,
        ],
    )(x.reshape(-1), pos_table.reshape(-1))
    return out_flat.reshape(B, S, D)


# speedup vs baseline: 2.8949x; 2.5569x over previous
import jax
import jax.numpy as jnp
from jax import lax
from jax.experimental import pallas as pl
from jax.experimental.pallas import tpu as pltpu
from jax.experimental.pallas import tpu_sc as plsc

_NC, _NS, _L = 2, 16, 16
_NW = _NC * _NS
_CH = 8
_B, _S, _D = 4, 2048, 4096


def _sc_body(x_hbm, tab_hbm, o_hbm, xb0, xb1, tbuf, ls0, ls1, ss0, ss1):
    rows_per_w = _S // _NW
    n_steps = (rows_per_w // _CH) * _B
    wid = lax.axis_index("c") * _NS + lax.axis_index("s")
    s_base = wid * rows_per_w

    xbufs = (xb0, xb1)
    lsems = (ls0, ls1)
    ssems = (ss0, ss1)

    def src_slice(t):
        c, b = t // _B, t % _B
        return (b, pl.ds(s_base + c * _CH, _CH))

    def load(t):
        b, sl = src_slice(t)
        return pltpu.make_async_copy(x_hbm.at[b, sl], xbufs[t % 2], lsems[t % 2])

    def store(t):
        b, sl = src_slice(t)
        return pltpu.make_async_copy(xbufs[t % 2], o_hbm.at[b, sl], ssems[t % 2])

    load(0).start()
    for t in range(n_steps):
        if t + 1 < n_steps:
            if t >= 1:
                store(t - 1).wait()
            load(t + 1).start()
        if t % _B == 0:
            pltpu.sync_copy(
                tab_hbm.at[pl.ds(s_base + (t // _B) * _CH, _CH)], tbuf)
        load(t).wait()

        xb = xbufs[t % 2]

        def row_body(r, _):
            def add_body(g, _):
                plsc.addupdate(xb.at[r, pl.ds(g * _L, _L)],
                               tbuf[r, pl.ds(g * _L, _L)])
                return 0

            return lax.fori_loop(0, _D // _L, add_body, 0, unroll=8)

        lax.fori_loop(0, _CH, row_body, 0)
        store(t).start()
    store(n_steps - 2).wait()
    store(n_steps - 1).wait()


def kernel(x, pos_table):
    B, S, D = x.shape
    return pl.kernel(
        _sc_body,
        out_type=jax.ShapeDtypeStruct((B, S, D), x.dtype),
        mesh=plsc.VectorSubcoreMesh(core_axis_name="c", subcore_axis_name="s"),
        scratch_types=[
            pltpu.VMEM((_CH, _D), jnp.float32),
            pltpu.VMEM((_CH, _D), jnp.float32),
            pltpu.VMEM((_CH, _D), jnp.float32),
            pltpu.SemaphoreType.DMA,
            pltpu.SemaphoreType.DMA,
            pltpu.SemaphoreType.DMA,
            pltpu.SemaphoreType.DMA,
        ],
    )(x, pos_table)
